# baseline (speedup 1.0000x reference)
import functools

import jax
import jax.numpy as jnp
from jax import lax
from jax.experimental import pallas as pl
from jax.experimental.pallas import tpu as pltpu
from jax.experimental.pallas import tpu_sc as plsc

PAD = 0
D_MODEL = 64
NUM_CORES = 2
NUM_SUBCORES = 16
NUM_WORKERS = NUM_CORES * NUM_SUBCORES


def _masks_body(tok_ref, pad_ref, seq_ref):
    pad_ref[...] = tok_ref[...] == PAD
    n = seq_ref.shape[0]
    row = lax.broadcasted_iota(jnp.int32, (n, n), 0)
    col = lax.broadcasted_iota(jnp.int32, (n, n), 1)
    seq_ref[...] = col > row


def _masks(tokens, bsz, seq_len):
    return pl.pallas_call(
        _masks_body,
        out_shape=(
            jax.ShapeDtypeStruct((bsz, seq_len), jnp.bool_),
            jax.ShapeDtypeStruct((seq_len, seq_len), jnp.bool_),
        ),
    )(tokens)


@functools.lru_cache(maxsize=None)
def _make_gather(B, chunk):
    b_per_w = B // NUM_WORKERS
    n_chunks = b_per_w // chunk
    mesh = plsc.VectorSubcoreMesh(core_axis_name="c", subcore_axis_name="s")

    @functools.partial(
        pl.kernel,
        mesh=mesh,
        out_type=jax.ShapeDtypeStruct((B, D_MODEL), jnp.float32),
        scratch_types=[
            pltpu.VMEM((chunk,), jnp.int32),
            pltpu.VMEM((chunk, D_MODEL), jnp.float32),
            pltpu.SemaphoreType.DMA,
        ],
        compiler_params=pltpu.CompilerParams(use_tc_tiling_on_sc=False),
    )
    def k(idx_hbm, table_hbm, out_hbm, idx_v, rows_v, sem):
        wid = lax.axis_index("s") * NUM_CORES + lax.axis_index("c")
        base = wid * b_per_w

        def body(j, carry):
            off = base + j * chunk
            pltpu.sync_copy(idx_hbm.at[pl.ds(off, chunk)], idx_v)
            pltpu.async_copy(table_hbm.at[idx_v], rows_v, sem).wait()
            pltpu.sync_copy(rows_v, out_hbm.at[pl.ds(off, chunk)])
            return carry

        lax.fori_loop(0, n_chunks, body, 0)

    return k


def kernel(tokens, table):
    bsz, seq_len = tokens.shape
    tok32 = tokens.astype(jnp.int32)
    idx_flat = tok32.reshape(-1)
    feats = _make_gather(idx_flat.shape[0], 128)(idx_flat, table)
    pad, seq = _masks(tok32, bsz, seq_len)
    return (
        feats.reshape(bsz, seq_len, D_MODEL),
        pad[:, None, None, :],
        seq,
    )


# speedup vs baseline: 1.0739x; 1.0739x over previous
import functools

import jax
import jax.numpy as jnp
from jax import lax
from jax.experimental import pallas as pl
from jax.experimental.pallas import tpu as pltpu
from jax.experimental.pallas import tpu_sc as plsc

PAD = 0
D_MODEL = 64
NUM_CORES = 2
NUM_SUBCORES = 16
NUM_WORKERS = NUM_CORES * NUM_SUBCORES


def _masks_body(tok_ref, pad_ref, seq_ref):
    pad_ref[...] = tok_ref[...] == PAD
    n = seq_ref.shape[0]
    row = lax.broadcasted_iota(jnp.int32, (n, n), 0)
    col = lax.broadcasted_iota(jnp.int32, (n, n), 1)
    seq_ref[...] = col > row


def _masks(tokens, bsz, seq_len):
    return pl.pallas_call(
        _masks_body,
        out_shape=(
            jax.ShapeDtypeStruct((bsz, seq_len), jnp.bool_),
            jax.ShapeDtypeStruct((seq_len, seq_len), jnp.bool_),
        ),
    )(tokens)


CHUNK = 128
K = 5


@functools.lru_cache(maxsize=None)
def _make_gather(B):
    b_per_w = B // NUM_WORKERS
    n_chunks = b_per_w // CHUNK
    n_groups = n_chunks // K
    assert n_chunks % K == 0 and n_groups % 2 == 0 and n_groups >= 4
    mesh = plsc.VectorSubcoreMesh(core_axis_name="c", subcore_axis_name="s")

    @functools.partial(
        pl.kernel,
        mesh=mesh,
        out_type=jax.ShapeDtypeStruct((B, D_MODEL), jnp.float32),
        scratch_types=[
            pltpu.VMEM((n_chunks, CHUNK), jnp.int32),
            pltpu.VMEM((2, K, CHUNK, D_MODEL), jnp.float32),
            pltpu.SemaphoreType.DMA,
            pltpu.SemaphoreType.DMA,
            pltpu.SemaphoreType.DMA,
        ],
        compiler_params=pltpu.CompilerParams(use_tc_tiling_on_sc=False),
    )
    def k(idx_hbm, table_hbm, out_hbm, idx_v, rows_v, gsem, ssem0, ssem1):
        wid = lax.axis_index("s") * NUM_CORES + lax.axis_index("c")
        rbase = wid * n_chunks
        obase = wid * b_per_w

        pltpu.sync_copy(idx_hbm.at[pl.ds(rbase, n_chunks)], idx_v)

        def issue_gathers(g, s):
            for i in range(K):
                pltpu.async_copy(
                    table_hbm.at[idx_v.at[g * K + i]], rows_v.at[s, i], gsem)

        def wait_gathers():
            for i in range(K):
                pltpu.make_async_copy(
                    table_hbm.at[pl.ds(0, CHUNK)], rows_v.at[0, i], gsem).wait()

        def issue_stores(g, s, sem):
            for i in range(K):
                off = obase + (g * K + i) * CHUNK
                pltpu.async_copy(
                    rows_v.at[s, i], out_hbm.at[pl.ds(off, CHUNK)], sem)

        def wait_stores(sem):
            for i in range(K):
                pltpu.make_async_copy(
                    table_hbm.at[pl.ds(0, CHUNK)], rows_v.at[0, i], sem).wait()

        issue_gathers(0, 0)
        wait_gathers()
        issue_stores(0, 0, ssem0)
        issue_gathers(1, 1)

        def body(g2, carry):
            g = 1 + 2 * g2
            wait_gathers()
            issue_stores(g, 1, ssem1)
            wait_stores(ssem0)
            issue_gathers(g + 1, 0)
            g = 2 + 2 * g2
            wait_gathers()
            issue_stores(g, 0, ssem0)
            wait_stores(ssem1)
            issue_gathers(g + 1, 1)
            return carry

        lax.fori_loop(0, (n_groups - 2) // 2, body, 0)

        wait_gathers()
        issue_stores(n_groups - 1, 1, ssem1)
        wait_stores(ssem0)
        wait_stores(ssem1)

    return k


def kernel(tokens, table):
    bsz, seq_len = tokens.shape
    tok32 = tokens.astype(jnp.int32)
    b_total = bsz * seq_len
    idx2d = tok32.reshape(b_total // CHUNK, CHUNK)
    feats = _make_gather(b_total)(idx2d, table)
    pad, seq = _masks(tok32, bsz, seq_len)
    return (
        feats.reshape(bsz, seq_len, D_MODEL),
        pad[:, None, None, :],
        seq,
    )


# speedup vs baseline: 1.0766x; 1.0025x over previous
import functools

import jax
import jax.numpy as jnp
from jax import lax
from jax.experimental import pallas as pl
from jax.experimental.pallas import tpu as pltpu
from jax.experimental.pallas import tpu_sc as plsc

PAD = 0
D_MODEL = 64
NUM_CORES = 2
NUM_SUBCORES = 16
NUM_WORKERS = NUM_CORES * NUM_SUBCORES

SEQ = 200
ROWS_PW = 32
R = 2
N_GROUPS = ROWS_PW // R


def _seq_mask_body(seq_ref):
    n = seq_ref.shape[0]
    row = lax.broadcasted_iota(jnp.int32, (n, n), 0)
    col = lax.broadcasted_iota(jnp.int32, (n, n), 1)
    seq_ref[...] = col > row


@functools.lru_cache(maxsize=None)
def _make_gather(bsz, seq_len):
    assert seq_len == SEQ and bsz == ROWS_PW * NUM_WORKERS
    mesh = plsc.VectorSubcoreMesh(core_axis_name="c", subcore_axis_name="s")

    @functools.partial(
        pl.kernel,
        mesh=mesh,
        out_type=(
            jax.ShapeDtypeStruct((bsz, SEQ, D_MODEL), jnp.float32),
            jax.ShapeDtypeStruct((bsz, SEQ), jnp.int32),
        ),
        scratch_types=[
            pltpu.VMEM((ROWS_PW, SEQ), jnp.int32),
            pltpu.VMEM((ROWS_PW, SEQ), jnp.int32),
            pltpu.VMEM((2, R, SEQ, D_MODEL), jnp.float32),
            pltpu.SemaphoreType.DMA,
            pltpu.SemaphoreType.DMA,
            pltpu.SemaphoreType.DMA,
        ],
        compiler_params=pltpu.CompilerParams(use_tc_tiling_on_sc=False),
    )
    def k(tok_hbm, table_hbm, out_hbm, pad_hbm, idx_v, mask_v, rows_v,
          gsem, ssem0, ssem1):
        wid = lax.axis_index("s") * NUM_CORES + lax.axis_index("c")
        b0 = wid * ROWS_PW

        pltpu.sync_copy(tok_hbm.at[pl.ds(b0, ROWS_PW), :], idx_v)

        def issue_gathers(g, s):
            for j in range(R):
                r = g * R + j
                pltpu.async_copy(
                    table_hbm.at[idx_v.at[r, pl.ds(0, 128)]],
                    rows_v.at[s, j, pl.ds(0, 128)], gsem)
                pltpu.async_copy(
                    table_hbm.at[idx_v.at[r, pl.ds(128, SEQ - 128)]],
                    rows_v.at[s, j, pl.ds(128, SEQ - 128)], gsem)

        def wait_gathers():
            for j in range(R):
                pltpu.make_async_copy(
                    table_hbm.at[pl.ds(0, 128)],
                    rows_v.at[0, j, pl.ds(0, 128)], gsem).wait()
                pltpu.make_async_copy(
                    table_hbm.at[pl.ds(0, SEQ - 128)],
                    rows_v.at[0, j, pl.ds(128, SEQ - 128)], gsem).wait()

        def issue_stores(g, s, sem):
            for j in range(R):
                pltpu.async_copy(rows_v.at[s, j], out_hbm.at[b0 + g * R + j],
                                 sem)

        def wait_stores(sem):
            for j in range(R):
                pltpu.make_async_copy(
                    table_hbm.at[pl.ds(0, SEQ)], rows_v.at[0, j], sem).wait()

        issue_gathers(0, 0)

        def mask_body(r, carry):
            for c in list(range(0, SEQ - 16, 16)) + [SEQ - 16]:
                v = idx_v[r, pl.ds(c, 16)]
                mask_v[r, pl.ds(c, 16)] = jnp.where(v == PAD, 1, 0)
            return carry

        lax.fori_loop(0, ROWS_PW, mask_body, 0)
        pltpu.sync_copy(mask_v, pad_hbm.at[pl.ds(b0, ROWS_PW), :])

        wait_gathers()
        issue_stores(0, 0, ssem0)
        issue_gathers(1, 1)

        def body(g2, carry):
            g = 1 + 2 * g2
            wait_gathers()
            issue_stores(g, 1, ssem1)
            wait_stores(ssem0)
            issue_gathers(g + 1, 0)
            g = 2 + 2 * g2
            wait_gathers()
            issue_stores(g, 0, ssem0)
            wait_stores(ssem1)
            issue_gathers(g + 1, 1)
            return carry

        lax.fori_loop(0, (N_GROUPS - 2) // 2, body, 0)

        wait_gathers()
        issue_stores(N_GROUPS - 1, 1, ssem1)
        wait_stores(ssem0)
        wait_stores(ssem1)

    return k


def kernel(tokens, table):
    bsz, seq_len = tokens.shape
    tok32 = tokens.astype(jnp.int32)
    feats, pad_i32 = _make_gather(bsz, seq_len)(tok32, table)
    seq = pl.pallas_call(
        _seq_mask_body,
        out_shape=jax.ShapeDtypeStruct((seq_len, seq_len), jnp.bool_),
    )()
    return feats, pad_i32.astype(jnp.bool_)[:, None, None, :], seq


# speedup vs baseline: 1.0788x; 1.0020x over previous
import functools

import jax
import jax.numpy as jnp
from jax import lax
from jax.experimental import pallas as pl
from jax.experimental.pallas import tpu as pltpu
from jax.experimental.pallas import tpu_sc as plsc

PAD = 0
D_MODEL = 64
NUM_CORES = 2
NUM_SUBCORES = 16
NUM_WORKERS = NUM_CORES * NUM_SUBCORES

IDX_PW = 6400
DESC = 128
K = 2
GROUP = K * DESC
N_GROUPS = IDX_PW // GROUP
NBUF = 5


def _masks_body(tok_ref, pad_ref, seq_ref):
    pad_ref[...] = tok_ref[...] == PAD
    n = seq_ref.shape[0]
    row = lax.broadcasted_iota(jnp.int32, (n, n), 0)
    col = lax.broadcasted_iota(jnp.int32, (n, n), 1)
    seq_ref[...] = col > row


@functools.lru_cache(maxsize=None)
def _make_gather(n_tok):
    assert n_tok == IDX_PW * NUM_WORKERS
    mesh = plsc.VectorSubcoreMesh(core_axis_name="c", subcore_axis_name="s")

    @functools.partial(
        pl.kernel,
        mesh=mesh,
        out_type=jax.ShapeDtypeStruct((n_tok, D_MODEL), jnp.float32),
        scratch_types=[
            pltpu.VMEM((IDX_PW // DESC, DESC), jnp.int32),
            pltpu.VMEM((NBUF, GROUP, D_MODEL), jnp.float32),
        ]
        + [pltpu.SemaphoreType.DMA] * (2 * NBUF),
        compiler_params=pltpu.CompilerParams(use_tc_tiling_on_sc=False),
    )
    def k(tok_hbm, table_hbm, out_hbm, idx_v, rows_v, *sems):
        gsem = sems[:NBUF]
        ssem = sems[NBUF:]
        wid = lax.axis_index("s") * NUM_CORES + lax.axis_index("c")
        base = wid * IDX_PW

        n_desc = IDX_PW // DESC
        pltpu.sync_copy(tok_hbm.at[pl.ds(wid * n_desc, n_desc)], idx_v)

        def issue_gather(g, s):
            for d in range(K):
                pltpu.async_copy(
                    table_hbm.at[idx_v.at[g * K + d]],
                    rows_v.at[s, pl.ds(d * DESC, DESC)], gsem[s])

        def wait_gather(s):
            for d in range(K):
                pltpu.make_async_copy(
                    table_hbm.at[pl.ds(0, DESC)],
                    rows_v.at[s, pl.ds(d * DESC, DESC)], gsem[s]).wait()

        def issue_store(g, s):
            pltpu.async_copy(rows_v.at[s],
                             out_hbm.at[pl.ds(base + g * GROUP, GROUP)],
                             ssem[s])

        def wait_store(s):
            pltpu.make_async_copy(
                table_hbm.at[pl.ds(0, GROUP)], rows_v.at[s], ssem[s]).wait()

        for g in range(NBUF - 1):
            issue_gather(g, g)

        wait_gather(0)
        issue_store(0, 0)
        issue_gather(NBUF - 1, NBUF - 1)

        def body(o, carry):
            for j in range(NBUF):
                g = (NBUF * o + 1) + j
                s = (1 + j) % NBUF
                t = j % NBUF
                wait_gather(s)
                issue_store(g, s)
                wait_store(t)
                issue_gather(g + NBUF - 1, t)
            return carry

        lax.fori_loop(0, (N_GROUPS - NBUF) // NBUF, body, 0)

        for j in range(NBUF - 1):
            g = N_GROUPS - NBUF + 1 + j
            s = g % NBUF
            wait_gather(s)
            issue_store(g, s)

        for s in range(NBUF):
            wait_store(s)

    return k


def kernel(tokens, table):
    bsz, seq_len = tokens.shape
    tok32 = tokens.astype(jnp.int32)
    feats = _make_gather(bsz * seq_len)(tok32.reshape(-1, DESC), table)
    pad, seq = pl.pallas_call(
        _masks_body,
        out_shape=(
            jax.ShapeDtypeStruct((bsz, seq_len), jnp.bool_),
            jax.ShapeDtypeStruct((seq_len, seq_len), jnp.bool_),
        ),
    )(tok32)
    return (feats.reshape(bsz, seq_len, D_MODEL),
            pad[:, None, None, :], seq)


# speedup vs baseline: 1.0799x; 1.0010x over previous
import functools

import jax
import jax.numpy as jnp
from jax import lax
from jax.experimental import pallas as pl
from jax.experimental.pallas import tpu as pltpu
from jax.experimental.pallas import tpu_sc as plsc

PAD = 0
D_MODEL = 64
NUM_CORES = 2
NUM_SUBCORES = 16
NUM_WORKERS = NUM_CORES * NUM_SUBCORES

IDX_PW = 6400
DESC = 128
K = 2
GROUP = K * DESC
N_GROUPS = IDX_PW // GROUP
NBUF = 5
SNBUF = 2


def _masks_body(tok_ref, pad_ref, seq_ref):
    pad_ref[...] = tok_ref[...] == PAD
    n = seq_ref.shape[0]
    row = lax.broadcasted_iota(jnp.int32, (n, n), 0)
    col = lax.broadcasted_iota(jnp.int32, (n, n), 1)
    seq_ref[...] = col > row


@functools.lru_cache(maxsize=None)
def _make_gather(n_tok):
    assert n_tok == IDX_PW * NUM_WORKERS
    mesh = plsc.VectorSubcoreMesh(core_axis_name="c", subcore_axis_name="s")

    @functools.partial(
        pl.kernel,
        mesh=mesh,
        out_type=jax.ShapeDtypeStruct((n_tok, D_MODEL), jnp.float32),
        scratch_types=[
            pltpu.VMEM((IDX_PW // DESC, DESC), jnp.int32),
            pltpu.VMEM((NBUF, GROUP, D_MODEL), jnp.float32),
            pltpu.VMEM_SHARED((NUM_SUBCORES, SNBUF, GROUP, D_MODEL),
                              jnp.float32),
        ]
        + [pltpu.SemaphoreType.DMA] * (NBUF + SNBUF),
        compiler_params=pltpu.CompilerParams(use_tc_tiling_on_sc=False),
    )
    def k(tok_hbm, table_hbm, out_hbm, idx_v, rows_v, shared, *sems):
        gsem = sems[:NBUF]
        ssem = sems[NBUF:]
        sid = lax.axis_index("s")
        wid = sid * NUM_CORES + lax.axis_index("c")
        base = wid * IDX_PW

        n_desc = IDX_PW // DESC
        pltpu.sync_copy(tok_hbm.at[pl.ds(wid * n_desc, n_desc)], idx_v)

        def issue_gather(g, s):
            for d in range(K):
                pltpu.async_copy(
                    table_hbm.at[idx_v.at[g * K + d]],
                    rows_v.at[s, pl.ds(d * DESC, DESC)], gsem[s])

        def wait_gather(s):
            for d in range(K):
                pltpu.make_async_copy(
                    table_hbm.at[pl.ds(0, DESC)],
                    rows_v.at[s, pl.ds(d * DESC, DESC)], gsem[s]).wait()

        def issue_store(g, s, u, first):
            if not first:
                wait_store(u)
            pltpu.sync_copy(rows_v.at[s], shared.at[sid, u])
            pltpu.async_copy(shared.at[sid, u],
                             out_hbm.at[pl.ds(base + g * GROUP, GROUP)],
                             ssem[u])

        def wait_store(u):
            pltpu.make_async_copy(
                table_hbm.at[pl.ds(0, GROUP)], shared.at[sid, u],
                ssem[u]).wait()

        for g in range(NBUF - 1):
            issue_gather(g, g)

        for g in range(NBUF):
            s = g % NBUF
            t = (g + NBUF - 1) % NBUF
            issue_gather(g + NBUF - 1, t)
            wait_gather(s)
            issue_store(g, s, g % SNBUF, g < SNBUF)

        def body(o, carry):
            for j in range(NBUF):
                g = NBUF * (o + 1) + j
                s = j
                t = (j + NBUF - 1) % NBUF
                issue_gather(g + NBUF - 1, t)
                wait_gather(s)
                issue_store(g, s, j % SNBUF, False)
            return carry

        lax.fori_loop(0, (N_GROUPS - 2 * NBUF) // NBUF, body, 0)

        for j in range(NBUF):
            g = N_GROUPS - NBUF + j
            s = g % NBUF
            if g + NBUF - 1 < N_GROUPS:
                issue_gather(g + NBUF - 1, (g + NBUF - 1) % NBUF)
            wait_gather(s)
            issue_store(g, s, g % SNBUF, False)

        for u in range(SNBUF):
            wait_store(u)

    return k


def kernel(tokens, table):
    bsz, seq_len = tokens.shape
    tok32 = tokens.astype(jnp.int32)
    feats = _make_gather(bsz * seq_len)(tok32.reshape(-1, DESC), table)
    pad, seq = pl.pallas_call(
        _masks_body,
        out_shape=(
            jax.ShapeDtypeStruct((bsz, seq_len), jnp.bool_),
            jax.ShapeDtypeStruct((seq_len, seq_len), jnp.bool_),
        ),
    )(tok32)
    return (feats.reshape(bsz, seq_len, D_MODEL),
            pad[:, None, None, :], seq)
